# baseline (speedup 1.0000x reference)
import dataclasses
import functools

import jax
import jax.numpy as jnp
from jax import lax
from jax.experimental import pallas as pl
from jax.experimental.pallas import tpu as pltpu
from jax.experimental.pallas import tpu_sc as plsc

ALPHA = 0.1
K_HOPS = 10

NC = 2
NS = 16
LANES = 16
EB = 128
CH = 104


def _sc_hop(h2, src4, dst3, val3, zeros, x2, n_nodes, dh, nb):
    rows_main = (n_nodes // NS) & ~7
    rem = n_nodes - rows_main * NS
    n_ch = rows_main // CH
    assert n_ch * CH == rows_main and CH <= EB and rem <= EB and nb % 6 == 0

    mesh = plsc.VectorSubcoreMesh(core_axis_name="c", subcore_axis_name="s")

    cp = pltpu.CompilerParams()
    fields = pltpu.CompilerParams.__dataclass_fields__
    if "needs_layout_passes" in fields:
        cp = dataclasses.replace(cp, needs_layout_passes=False)
    if "use_tc_tiling_on_sc" in fields:
        cp = dataclasses.replace(cp, use_tc_tiling_on_sc=False)

    @functools.partial(
        pl.kernel,
        out_type=jax.ShapeDtypeStruct((NC * n_nodes, dh), jnp.float32),
        mesh=mesh,
        compiler_params=cp,
        scratch_types=[
            pltpu.VMEM((nb, EB), jnp.int32),
            pltpu.VMEM((nb, EB), jnp.int32),
            pltpu.VMEM((nb, EB), jnp.float32),
            pltpu.VMEM((3, EB, dh), jnp.float32),
            pltpu.VMEM_SHARED((n_nodes, dh), jnp.float32),
            pltpu.SemaphoreType.DMA,
            pltpu.SemaphoreType.DMA,
            pltpu.SemaphoreType.DMA,
            pltpu.SemaphoreType.DMA,
            pltpu.SemaphoreType.DMA,
            pltpu.SemaphoreType.DMA,
            pltpu.SemaphoreType.DMA,
        ],
    )
    def prop(h_hbm, src_hbm, dst_hbm, val_hbm, zero_hbm, x2_hbm, out_hbm,
             src_all, dst_all, val_all, rows_v, acc_sh, sem_i,
             sem_g0, sem_g1, sem_g2, sem_s0, sem_s1, sem_s2):
        cid = lax.axis_index("c")
        sid = lax.axis_index("s")
        wid = cid * NS + sid
        sem_g = (sem_g0, sem_g1, sem_g2)
        sem_s = (sem_s0, sem_s1, sem_s2)

        pltpu.async_copy(src_hbm.at[wid], src_all, sem_i)
        pltpu.async_copy(dst_hbm.at[sid], dst_all, sem_i)
        pltpu.async_copy(val_hbm.at[sid], val_all, sem_i)

        r0 = sid * rows_main
        pltpu.sync_copy(zero_hbm.at[pl.ds(r0, rows_main)],
                        acc_sh.at[pl.ds(r0, rows_main)])
        if rem:
            @pl.when(sid == NS - 1)
            def _():
                pltpu.sync_copy(zero_hbm.at[pl.ds(rows_main * NS, rem)],
                                acc_sh.at[pl.ds(rows_main * NS, rem)])

        pltpu.make_async_copy(src_hbm.at[wid], src_all, sem_i).wait()
        pltpu.make_async_copy(dst_hbm.at[sid], dst_all, sem_i).wait()
        pltpu.make_async_copy(val_hbm.at[sid], val_all, sem_i).wait()

        pltpu.async_copy(h_hbm.at[src_all.at[0]], rows_v.at[0], sem_g0)

        plsc.subcore_barrier()

        def substep(k, p):
            nxt = (p + 1) % 3
            pltpu.make_async_copy(
                h_hbm.at[src_all.at[k]], rows_v.at[p], sem_g[p]).wait()

            @pl.when(k >= 2)
            def _():
                pltpu.make_async_copy(
                    rows_v.at[nxt], acc_sh.at[dst_all.at[k - 2]],
                    sem_s[nxt]).wait()

            @pl.when(k + 1 < nb)
            def _():
                pltpu.async_copy(
                    h_hbm.at[src_all.at[k + 1]], rows_v.at[nxt], sem_g[nxt])

            @pl.loop(0, EB, step=2)
            def _(r):
                for dr in range(2):
                    rr = r + dr
                    vv = plsc.load_gather(
                        val_all, [jnp.full((LANES,), k, dtype=jnp.int32),
                                  jnp.full((LANES,), rr, dtype=jnp.int32)])
                    for c in range(dh // LANES):
                        sl = pl.ds(c * LANES, LANES)
                        rows_v[p, rr, sl] = rows_v[p, rr, sl] * vv

            pltpu.async_copy(rows_v.at[p], acc_sh.at[dst_all.at[k]],
                             sem_s[p], add=True)

        @pl.loop(0, nb // 3)
        def _(i):
            substep(3 * i, 0)
            substep(3 * i + 1, 1)
            substep(3 * i + 2, 2)

        pltpu.make_async_copy(
            rows_v.at[(nb - 2) % 3], acc_sh.at[dst_all.at[nb - 2]],
            sem_s[(nb - 2) % 3]).wait()
        pltpu.make_async_copy(
            rows_v.at[(nb - 1) % 3], acc_sh.at[dst_all.at[nb - 1]],
            sem_s[(nb - 1) % 3]).wait()

        plsc.subcore_barrier()

        def combine_rows(row0, nrows):
            a_v = rows_v.at[0, pl.ds(0, nrows)]
            x_v = rows_v.at[1, pl.ds(0, nrows)]
            pltpu.sync_copy(acc_sh.at[pl.ds(row0, nrows)], a_v)
            pltpu.sync_copy(x2_hbm.at[pl.ds(cid * n_nodes + row0, nrows)], x_v)

            @pl.loop(0, nrows)
            def _(r):
                for c in range(dh // LANES):
                    sl = pl.ds(c * LANES, LANES)
                    rows_v[0, r, sl] = ((1.0 - ALPHA) * rows_v[0, r, sl]
                                        + ALPHA * rows_v[1, r, sl])

            pltpu.sync_copy(
                a_v, out_hbm.at[pl.ds(cid * n_nodes + row0, nrows)])

        @pl.loop(0, n_ch)
        def _(j):
            combine_rows(r0 + j * CH, CH)

        if rem:
            @pl.when(sid == NS - 1)
            def _():
                combine_rows(rows_main * NS, rem)

    return prop(h2, src4, dst3, val3, zeros, x2)


def kernel(x, edge_index, adj_values):
    n_nodes, d = x.shape
    dh = d // NC
    dst = edge_index[0]
    src = edge_index[1]
    e = dst.shape[0]

    nb = -(-e // (NS * EB))
    nb += (-nb) % 6
    e_pad = nb * EB * NS
    pad = e_pad - e
    if pad:
        src = jnp.concatenate([src, jnp.zeros((pad,), src.dtype)])
        dst = jnp.concatenate([dst, jnp.zeros((pad,), dst.dtype)])
        adj = jnp.concatenate([adj_values, jnp.zeros((pad,), adj_values.dtype)])
    else:
        adj = adj_values
    src3 = src.reshape(NS, nb, EB)
    src4 = jnp.concatenate([src3, src3 + n_nodes], axis=0)
    dst3 = dst.reshape(NS, nb, EB)
    val3 = adj.reshape(NS, nb, EB)
    zeros = jnp.zeros((n_nodes, dh), jnp.float32)

    x2 = jnp.concatenate([x[:, :dh], x[:, dh:]], axis=0)

    h2 = x2
    for _ in range(K_HOPS):
        h2 = _sc_hop(h2, src4, dst3, val3, zeros, x2, n_nodes, dh, nb)

    return jnp.concatenate([h2[:n_nodes], h2[n_nodes:]], axis=1)
